# Initial kernel scaffold; baseline (speedup 1.0000x reference)
import jax
import jax.numpy as jnp
from jax.experimental import pallas as pl


def kernel(points):
    raise NotImplementedError("write your pallas kernel here")


# speedup vs baseline: 9.2616x; 9.2616x over previous
import functools

import jax
import jax.numpy as jnp
from jax import lax
from jax.experimental import pallas as pl
from jax.experimental.pallas import tpu as pltpu
from jax.experimental.pallas import tpu_sc as plsc

_LANES = 16


def _fps_body(N, K, NC, pts_hbm, first_hbm, out_hbm, xv, yv, zv, dv, fv, ob):
    c = lax.axis_index("c")
    s = lax.axis_index("s")
    b = s * NC + c

    pltpu.sync_copy(pts_hbm.at[0, b], xv)
    pltpu.sync_copy(pts_hbm.at[1, b], yv)
    pltpu.sync_copy(pts_hbm.at[2, b], zv)
    pltpu.sync_copy(first_hbm, fv)

    lane = lax.broadcasted_iota(jnp.int32, (_LANES,), 0)
    out_mask = lane < 3

    bvec = jnp.full((_LANES,), b, dtype=jnp.int32)
    fvec = plsc.load_gather(fv, [bvec])
    lx = plsc.load_gather(xv, [fvec])
    ly = plsc.load_gather(yv, [fvec])
    lz = plsc.load_gather(zv, [fvec])
    out0 = jnp.where(lane == 0, lx, jnp.where(lane == 1, ly, lz))
    plsc.store_scatter(ob, [lane], out0, mask=out_mask)

    def init_chunk(i, carry):
        dv[pl.ds(i * _LANES, _LANES)] = jnp.full((_LANES,), 1e10, jnp.float32)
        return carry

    lax.fori_loop(0, N // _LANES, init_chunk, 0, unroll=8)

    def step(t, carry):
        clx, cly, clz = carry

        def chunk(i, rc):
            rmax, ridx = rc
            base = i * _LANES
            x = xv[pl.ds(base, _LANES)]
            y = yv[pl.ds(base, _LANES)]
            z = zv[pl.ds(base, _LANES)]
            dx = x - clx
            dy = y - cly
            dz = z - clz
            dist = (dx * dx + dz * dz) + dy * dy
            d = jnp.minimum(dv[pl.ds(base, _LANES)], dist)
            dv[pl.ds(base, _LANES)] = d
            take = d > rmax
            rmax = jnp.where(take, d, rmax)
            ridx = jnp.where(take, base + lane, ridx)
            return rmax, ridx

        rmax0 = jnp.full((_LANES,), -1.0, jnp.float32)
        ridx0 = jnp.zeros((_LANES,), jnp.int32)
        rmax, ridx = lax.fori_loop(0, N // _LANES, chunk, (rmax0, ridx0),
                                   unroll=8)

        m = jnp.max(rmax)
        cand = jnp.where(rmax == m, ridx, jnp.int32(2**31 - 1))
        idx = jnp.min(cand)

        idxvec = jnp.full((_LANES,), idx, dtype=jnp.int32)
        nlx = plsc.load_gather(xv, [idxvec])
        nly = plsc.load_gather(yv, [idxvec])
        nlz = plsc.load_gather(zv, [idxvec])
        outv = jnp.where(lane == 0, nlx, jnp.where(lane == 1, nly, nlz))
        plsc.store_scatter(ob, [3 * (t + 1) + lane], outv, mask=out_mask)
        return nlx, nly, nlz

    lax.fori_loop(0, K - 1, step, (lx, ly, lz))
    pltpu.sync_copy(ob, out_hbm.at[b])


def kernel(points):
    B, N, D = points.shape
    K = min(1024, N)
    assert D == 3, "kernel specialized to 3-D points"

    info = plsc.get_sparse_core_info()
    NC, NS = info.num_cores, info.num_subcores
    assert NC * NS == B, (NC, NS, B)

    pts_t = jnp.transpose(points, (2, 0, 1))
    first = jax.random.randint(jax.random.key(1), (B,), 0, N).astype(jnp.int32)

    mesh = plsc.VectorSubcoreMesh(core_axis_name="c", subcore_axis_name="s")
    fps = pl.kernel(
        functools.partial(_fps_body, N, K, NC),
        mesh=mesh,
        compiler_params=pltpu.CompilerParams(needs_layout_passes=False),
        out_type=jax.ShapeDtypeStruct((B, K * 3), jnp.float32),
        scratch_types=[
            pltpu.VMEM((N,), jnp.float32),
            pltpu.VMEM((N,), jnp.float32),
            pltpu.VMEM((N,), jnp.float32),
            pltpu.VMEM((N,), jnp.float32),
            pltpu.VMEM((B,), jnp.int32),
            pltpu.VMEM((K * 3,), jnp.float32),
        ],
    )
    out = fps(pts_t, first)
    return out.reshape(B, K, 3)


# speedup vs baseline: 18.6893x; 2.0179x over previous
import functools

import jax
import jax.numpy as jnp
from jax import lax
from jax.experimental import pallas as pl
from jax.experimental.pallas import tpu as pltpu
from jax.experimental.pallas import tpu_sc as plsc

_LANES = 16
_G = 32
_IMAX = 2**31 - 1


def _bcast_max(v):
    pre = plsc.cummax(v)
    suf = jnp.flip(plsc.cummax(jnp.flip(v, 0)), 0)
    return jnp.maximum(pre, suf)


def _bcast_min_i32(v):
    return -_bcast_max(-v)


def _fps_body(N, K, NG, NC, pts_hbm, oidx_hbm, ctr_hbm, rad_hbm, first_hbm,
              out_hbm, xv, yv, zv, dv, ov, cxv, cyv, czv, rv, wv, gmv, actv,
              fv, ob):
    c = lax.axis_index("c")
    s = lax.axis_index("s")
    b = s * NC + c

    pltpu.sync_copy(pts_hbm.at[0, b], xv)
    pltpu.sync_copy(pts_hbm.at[1, b], yv)
    pltpu.sync_copy(pts_hbm.at[2, b], zv)
    pltpu.sync_copy(oidx_hbm.at[b], ov)
    pltpu.sync_copy(ctr_hbm.at[0, b], cxv)
    pltpu.sync_copy(ctr_hbm.at[1, b], cyv)
    pltpu.sync_copy(ctr_hbm.at[2, b], czv)
    pltpu.sync_copy(rad_hbm.at[b], rv)
    pltpu.sync_copy(first_hbm, fv)

    lane = lax.broadcasted_iota(jnp.int32, (_LANES,), 0)
    out_mask = lane < 3
    lane0 = lane == 0

    def init_d(i, carry):
        dv[pl.ds(i * _LANES, _LANES)] = jnp.full((_LANES,), 1e10, jnp.float32)
        return carry

    lax.fori_loop(0, N // _LANES, init_d, 0, unroll=8)

    def init_g(i, carry):
        gmv[pl.ds(i * _LANES, _LANES)] = jnp.full((_LANES,), 1e10, jnp.float32)
        wv[pl.ds(i * _LANES, _LANES)] = jnp.full((_LANES,), 3e38, jnp.float32)
        return carry

    lax.fori_loop(0, NG // _LANES, init_g, 0, unroll=4)

    bvec = jnp.full((_LANES,), b, dtype=jnp.int32)
    fvec = plsc.load_gather(fv, [bvec])
    lx = plsc.load_gather(xv, [fvec])
    ly = plsc.load_gather(yv, [fvec])
    lz = plsc.load_gather(zv, [fvec])
    out0 = jnp.where(lane == 0, lx, jnp.where(lane == 1, ly, lz))
    plsc.store_scatter(ob, [lane], out0, mask=out_mask)

    def step(t, carry):
        clx, cly, clz = carry

        def ph1(i, offv):
            gbase = i * _LANES
            cx = cxv[pl.ds(gbase, _LANES)]
            cy = cyv[pl.ds(gbase, _LANES)]
            cz = czv[pl.ds(gbase, _LANES)]
            w = wv[pl.ds(gbase, _LANES)]
            dcx = cx - clx
            dcy = cy - cly
            dcz = cz - clz
            dc = (dcx * dcx + dcz * dcz) + dcy * dcy
            act = dc < w
            pos = offv + plsc.cumsum(jnp.where(act, 1, 0)) - 1
            plsc.store_scatter(actv, [pos], gbase + lane, mask=act)
            return offv + plsc.all_reduce_population_count(act)

        offv = lax.fori_loop(0, NG // _LANES, ph1,
                             jnp.zeros((_LANES,), jnp.int32))
        na = jnp.max(offv)

        def ph2(j, _):
            gidv = plsc.load_gather(actv, [jnp.full((_LANES,), j, jnp.int32)])
            gid = jnp.max(gidv)
            base = gid * _G
            gm = jnp.full((_LANES,), -1.0, jnp.float32)
            for u in range(_G // _LANES):
                sl = pl.ds(base + u * _LANES, _LANES)
                dx = xv[sl] - clx
                dy = yv[sl] - cly
                dz = zv[sl] - clz
                dist = (dx * dx + dz * dz) + dy * dy
                d = jnp.minimum(dv[sl], dist)
                dv[sl] = d
                gm = jnp.maximum(gm, d)
            gmb = _bcast_max(gm)
            yi = jnp.int32(0x5F3759DF) - (plsc.bitcast(gmb, jnp.int32) >> 1)
            y0 = plsc.bitcast(yi, jnp.float32)
            y1 = y0 * (1.5 - 0.5 * gmb * y0 * y0)
            y2 = y1 * (1.5 - 0.5 * gmb * y1 * y1)
            sq = jnp.maximum(gmb * y2 * 1.0001, 1.2e-19)
            rg = plsc.load_gather(rv, [gidv])
            w = (rg + sq) * (rg + sq) * 1.0001
            plsc.store_scatter(gmv, [gidv], gmb, mask=lane0)
            plsc.store_scatter(wv, [gidv], w, mask=lane0)
            return 0

        plsc.parallel_loop(0, na, 1, carry=jnp.int32(0))(ph2)

        def ph3(i, rm):
            return jnp.maximum(rm, gmv[pl.ds(i * _LANES, _LANES)])

        rm = lax.fori_loop(0, NG // _LANES, ph3,
                           jnp.full((_LANES,), -1.0, jnp.float32), unroll=4)
        mv = _bcast_max(rm)

        def ph3b(i, offv):
            gbase = i * _LANES
            cand = gmv[pl.ds(gbase, _LANES)] == mv
            pos = offv + plsc.cumsum(jnp.where(cand, 1, 0)) - 1
            plsc.store_scatter(actv, [pos], gbase + lane, mask=cand)
            return offv + plsc.all_reduce_population_count(cand)

        offv = lax.fori_loop(0, NG // _LANES, ph3b,
                             jnp.zeros((_LANES,), jnp.int32))
        nc = jnp.max(offv)

        def ph3c(j, best):
            bo, bp = best
            gidv = plsc.load_gather(actv, [jnp.full((_LANES,), j, jnp.int32)])
            gid = jnp.max(gidv)
            base = gid * _G
            o_lane = jnp.full((_LANES,), _IMAX, jnp.int32)
            p_lane = jnp.full((_LANES,), _IMAX, jnp.int32)
            for u in range(_G // _LANES):
                sl = pl.ds(base + u * _LANES, _LANES)
                hit = dv[sl] == mv
                oo = jnp.where(hit, ov[sl], _IMAX)
                take = oo < o_lane
                o_lane = jnp.where(take, oo, o_lane)
                p_lane = jnp.where(take, base + u * _LANES + lane, p_lane)
            obst = _bcast_min_i32(o_lane)
            pbst = _bcast_min_i32(jnp.where(o_lane == obst, p_lane, _IMAX))
            take = obst < bo
            return (jnp.where(take, obst, bo), jnp.where(take, pbst, bp))

        _, bestp = lax.fori_loop(
            0, nc, ph3c,
            (jnp.full((_LANES,), _IMAX, jnp.int32), jnp.full((_LANES,), _IMAX, jnp.int32)))

        nlx = plsc.load_gather(xv, [bestp])
        nly = plsc.load_gather(yv, [bestp])
        nlz = plsc.load_gather(zv, [bestp])
        outv = jnp.where(lane == 0, nlx, jnp.where(lane == 1, nly, nlz))
        plsc.store_scatter(ob, [3 * (t + 1) + lane], outv, mask=out_mask)
        return nlx, nly, nlz

    lax.fori_loop(0, K - 1, step, (lx, ly, lz))
    pltpu.sync_copy(ob, out_hbm.at[b])


def _morton(q):
    def spread(x):
        x = (x | (x << 16)) & 0x030000FF
        x = (x | (x << 8)) & 0x0300F00F
        x = (x | (x << 4)) & 0x030C30C3
        x = (x | (x << 2)) & 0x09249249
        return x

    return (spread(q[..., 0]) << 2) | (spread(q[..., 1]) << 1) | spread(q[..., 2])


def kernel(points):
    B, N, D = points.shape
    K = min(1024, N)
    NG = N // _G
    assert D == 3, "kernel specialized to 3-D points"

    info = plsc.get_sparse_core_info()
    NC, NS = info.num_cores, info.num_subcores
    assert NC * NS == B, (NC, NS, B)

    lo = jnp.min(points, axis=1, keepdims=True)
    hi = jnp.max(points, axis=1, keepdims=True)
    q = jnp.clip(((points - lo) / (hi - lo + 1e-30) * 1024).astype(jnp.int32),
                 0, 1023)
    perm = jnp.argsort(_morton(q), axis=-1).astype(jnp.int32)
    ps = jnp.take_along_axis(points, perm[:, :, None], axis=1)
    inv = jnp.zeros((B, N), jnp.int32)
    inv = inv.at[jnp.arange(B)[:, None], perm].set(
        jnp.broadcast_to(jnp.arange(N, dtype=jnp.int32), (B, N)))

    grp = ps.reshape(B, NG, _G, 3)
    ctr = grp.mean(axis=2)
    rad = jnp.sqrt(((grp - ctr[:, :, None, :]) ** 2).sum(-1).max(axis=2))
    rad = rad * 1.001 + 1e-30

    first = jax.random.randint(jax.random.key(1), (B,), 0, N).astype(jnp.int32)
    first_p = jnp.take_along_axis(inv, first[:, None], axis=1)[:, 0]

    pts_t = jnp.transpose(ps, (2, 0, 1))
    ctr_t = jnp.transpose(ctr, (2, 0, 1))

    mesh = plsc.VectorSubcoreMesh(core_axis_name="c", subcore_axis_name="s")
    fps = pl.kernel(
        functools.partial(_fps_body, N, K, NG, NC),
        mesh=mesh,
        compiler_params=pltpu.CompilerParams(needs_layout_passes=False),
        out_type=jax.ShapeDtypeStruct((B, K * 3), jnp.float32),
        scratch_types=[
            pltpu.VMEM((N,), jnp.float32),
            pltpu.VMEM((N,), jnp.float32),
            pltpu.VMEM((N,), jnp.float32),
            pltpu.VMEM((N,), jnp.float32),
            pltpu.VMEM((N,), jnp.int32),
            pltpu.VMEM((NG,), jnp.float32),
            pltpu.VMEM((NG,), jnp.float32),
            pltpu.VMEM((NG,), jnp.float32),
            pltpu.VMEM((NG,), jnp.float32),
            pltpu.VMEM((NG,), jnp.float32),
            pltpu.VMEM((NG,), jnp.float32),
            pltpu.VMEM((NG + _LANES,), jnp.int32),
            pltpu.VMEM((B,), jnp.int32),
            pltpu.VMEM((K * 3,), jnp.float32),
        ],
    )
    out = fps(pts_t, perm, ctr_t, rad, first_p)
    return out.reshape(B, K, 3)


# speedup vs baseline: 34.4981x; 1.8459x over previous
import functools

import jax
import jax.numpy as jnp
from jax import lax
from jax.experimental import pallas as pl
from jax.experimental.pallas import tpu as pltpu
from jax.experimental.pallas import tpu_sc as plsc

_LANES = 16
_G = 32
_IMAX = 2**31 - 1


def _bcast_max(v):
    pre = plsc.cummax(v)
    suf = jnp.flip(plsc.cummax(jnp.flip(v, 0)), 0)
    return jnp.maximum(pre, suf)


def _bcast_min_i32(v):
    return -_bcast_max(-v)


def _fps_body(N, K, NG, NC, pts_hbm, oidx_hbm, ctr_hbm, rad_hbm, first_hbm,
              out_hbm, xv, yv, zv, dv, ov, cxv, cyv, czv, rv, wv, gmv, actv,
              fv, ob):
    c = lax.axis_index("c")
    s = lax.axis_index("s")
    b = s * NC + c

    pltpu.sync_copy(pts_hbm.at[0, b], xv)
    pltpu.sync_copy(pts_hbm.at[1, b], yv)
    pltpu.sync_copy(pts_hbm.at[2, b], zv)
    pltpu.sync_copy(oidx_hbm.at[b], ov)
    pltpu.sync_copy(ctr_hbm.at[0, b], cxv)
    pltpu.sync_copy(ctr_hbm.at[1, b], cyv)
    pltpu.sync_copy(ctr_hbm.at[2, b], czv)
    pltpu.sync_copy(rad_hbm.at[b], rv)
    pltpu.sync_copy(first_hbm, fv)

    lane = lax.broadcasted_iota(jnp.int32, (_LANES,), 0)
    out_mask = lane < 3
    lane0 = lane == 0

    def init_d(i, carry):
        dv[pl.ds(i * _LANES, _LANES)] = jnp.full((_LANES,), 1e10, jnp.float32)
        return carry

    lax.fori_loop(0, N // _LANES, init_d, 0, unroll=8)

    def init_g(i, carry):
        gmv[pl.ds(i * _LANES, _LANES)] = jnp.full((_LANES,), 1e10, jnp.float32)
        wv[pl.ds(i * _LANES, _LANES)] = jnp.full((_LANES,), 3e38, jnp.float32)
        return carry

    lax.fori_loop(0, NG // _LANES, init_g, 0, unroll=4)

    bvec = jnp.full((_LANES,), b, dtype=jnp.int32)
    fvec = plsc.load_gather(fv, [bvec])
    lx = plsc.load_gather(xv, [fvec])
    ly = plsc.load_gather(yv, [fvec])
    lz = plsc.load_gather(zv, [fvec])
    out0 = jnp.where(lane == 0, lx, jnp.where(lane == 1, ly, lz))
    plsc.store_scatter(ob, [lane], out0, mask=out_mask)

    def step(t, carry):
        clx, cly, clz = carry

        def ph1(i, offv):
            gbase = i * _LANES
            cx = cxv[pl.ds(gbase, _LANES)]
            cy = cyv[pl.ds(gbase, _LANES)]
            cz = czv[pl.ds(gbase, _LANES)]
            w = wv[pl.ds(gbase, _LANES)]
            dcx = cx - clx
            dcy = cy - cly
            dcz = cz - clz
            dc = (dcx * dcx + dcz * dcz) + dcy * dcy
            act = dc < w
            pos = offv + plsc.cumsum(jnp.where(act, 1, 0)) - 1
            plsc.store_scatter(actv, [pos], gbase + lane, mask=act)
            return offv + plsc.all_reduce_population_count(act)

        offv = plsc.parallel_loop(
            0, NG // _LANES, 1, unroll=4,
            carry=jnp.zeros((_LANES,), jnp.int32))(ph1)
        na = jnp.max(offv)

        def ph2(j):
            gidv = plsc.load_gather(actv, [jnp.full((_LANES,), j, jnp.int32)])
            basev = gidv * _G + lane
            gm = jnp.full((_LANES,), -1.0, jnp.float32)
            for u in range(_G // _LANES):
                idxv = basev + u * _LANES
                dx = plsc.load_gather(xv, [idxv]) - clx
                dy = plsc.load_gather(yv, [idxv]) - cly
                dz = plsc.load_gather(zv, [idxv]) - clz
                dist = (dx * dx + dz * dz) + dy * dy
                d = jnp.minimum(plsc.load_gather(dv, [idxv]), dist)
                plsc.store_scatter(dv, [idxv], d)
                gm = jnp.maximum(gm, d)
            gmb = _bcast_max(gm)
            yi = jnp.int32(0x5F3759DF) - (plsc.bitcast(gmb, jnp.int32) >> 1)
            y0 = plsc.bitcast(yi, jnp.float32)
            y1 = y0 * (1.5 - 0.5 * gmb * y0 * y0)
            y2 = y1 * (1.5 - 0.5 * gmb * y1 * y1)
            sq = jnp.maximum(gmb * y2 * 1.0001, 1.2e-19)
            rg = plsc.load_gather(rv, [gidv])
            w = (rg + sq) * (rg + sq) * 1.0001
            plsc.store_scatter(gmv, [gidv], gmb, mask=lane0)
            plsc.store_scatter(wv, [gidv], w, mask=lane0)

        plsc.parallel_loop(0, na, 1)(ph2)

        def ph3(i, rm):
            return jnp.maximum(rm, gmv[pl.ds(i * _LANES, _LANES)])

        rm = lax.fori_loop(0, NG // _LANES, ph3,
                           jnp.full((_LANES,), -1.0, jnp.float32), unroll=4)
        mv = _bcast_max(rm)

        def ph3b(i, offv):
            gbase = i * _LANES
            cand = gmv[pl.ds(gbase, _LANES)] == mv
            pos = offv + plsc.cumsum(jnp.where(cand, 1, 0)) - 1
            plsc.store_scatter(actv, [pos], gbase + lane, mask=cand)
            return offv + plsc.all_reduce_population_count(cand)

        offv = lax.fori_loop(0, NG // _LANES, ph3b,
                             jnp.zeros((_LANES,), jnp.int32))
        nc = jnp.max(offv)

        def ph3c(j, best):
            bo, bp = best
            gidv = plsc.load_gather(actv, [jnp.full((_LANES,), j, jnp.int32)])
            basev = gidv * _G + lane
            o_lane = jnp.full((_LANES,), _IMAX, jnp.int32)
            p_lane = jnp.full((_LANES,), _IMAX, jnp.int32)
            for u in range(_G // _LANES):
                idxv = basev + u * _LANES
                hit = plsc.load_gather(dv, [idxv]) == mv
                oo = jnp.where(hit, plsc.load_gather(ov, [idxv]), _IMAX)
                take = oo < o_lane
                o_lane = jnp.where(take, oo, o_lane)
                p_lane = jnp.where(take, idxv, p_lane)
            obst = _bcast_min_i32(o_lane)
            pbst = _bcast_min_i32(jnp.where(o_lane == obst, p_lane, _IMAX))
            take = obst < bo
            return (jnp.where(take, obst, bo), jnp.where(take, pbst, bp))

        _, bestp = lax.fori_loop(
            0, nc, ph3c,
            (jnp.full((_LANES,), _IMAX, jnp.int32), jnp.full((_LANES,), _IMAX, jnp.int32)))

        nlx = plsc.load_gather(xv, [bestp])
        nly = plsc.load_gather(yv, [bestp])
        nlz = plsc.load_gather(zv, [bestp])
        outv = jnp.where(lane == 0, nlx, jnp.where(lane == 1, nly, nlz))
        plsc.store_scatter(ob, [3 * (t + 1) + lane], outv, mask=out_mask)
        return nlx, nly, nlz

    lax.fori_loop(0, K - 1, step, (lx, ly, lz))
    pltpu.sync_copy(ob, out_hbm.at[b])


def _morton(q):
    def spread(x):
        x = (x | (x << 16)) & 0x030000FF
        x = (x | (x << 8)) & 0x0300F00F
        x = (x | (x << 4)) & 0x030C30C3
        x = (x | (x << 2)) & 0x09249249
        return x

    return (spread(q[..., 0]) << 2) | (spread(q[..., 1]) << 1) | spread(q[..., 2])


def kernel(points):
    B, N, D = points.shape
    K = min(1024, N)
    NG = N // _G
    assert D == 3, "kernel specialized to 3-D points"

    info = plsc.get_sparse_core_info()
    NC, NS = info.num_cores, info.num_subcores
    assert NC * NS == B, (NC, NS, B)

    lo = jnp.min(points, axis=1, keepdims=True)
    hi = jnp.max(points, axis=1, keepdims=True)
    q = jnp.clip(((points - lo) / (hi - lo + 1e-30) * 1024).astype(jnp.int32),
                 0, 1023)
    perm = jnp.argsort(_morton(q), axis=-1).astype(jnp.int32)
    ps = jnp.take_along_axis(points, perm[:, :, None], axis=1)

    grp = ps.reshape(B, NG, _G, 3)
    ctr = grp.mean(axis=2)
    rad = jnp.sqrt(((grp - ctr[:, :, None, :]) ** 2).sum(-1).max(axis=2))
    rad = rad * 1.001 + 1e-30

    first = jax.random.randint(jax.random.key(1), (B,), 0, N).astype(jnp.int32)
    first_p = jnp.argmax(perm == first[:, None], axis=-1).astype(jnp.int32)

    pts_t = jnp.transpose(ps, (2, 0, 1))
    ctr_t = jnp.transpose(ctr, (2, 0, 1))

    mesh = plsc.VectorSubcoreMesh(core_axis_name="c", subcore_axis_name="s")
    fps = pl.kernel(
        functools.partial(_fps_body, N, K, NG, NC),
        mesh=mesh,
        compiler_params=pltpu.CompilerParams(needs_layout_passes=False),
        out_type=jax.ShapeDtypeStruct((B, K * 3), jnp.float32),
        scratch_types=[
            pltpu.VMEM((N,), jnp.float32),
            pltpu.VMEM((N,), jnp.float32),
            pltpu.VMEM((N,), jnp.float32),
            pltpu.VMEM((N,), jnp.float32),
            pltpu.VMEM((N,), jnp.int32),
            pltpu.VMEM((NG,), jnp.float32),
            pltpu.VMEM((NG,), jnp.float32),
            pltpu.VMEM((NG,), jnp.float32),
            pltpu.VMEM((NG,), jnp.float32),
            pltpu.VMEM((NG,), jnp.float32),
            pltpu.VMEM((NG,), jnp.float32),
            pltpu.VMEM((NG + _LANES,), jnp.int32),
            pltpu.VMEM((B,), jnp.int32),
            pltpu.VMEM((K * 3,), jnp.float32),
        ],
    )
    out = fps(pts_t, perm, ctr_t, rad, first_p)
    return out.reshape(B, K, 3)


# speedup vs baseline: 38.0309x; 1.1024x over previous
import functools

import jax
import jax.numpy as jnp
from jax import lax
from jax.experimental import pallas as pl
from jax.experimental.pallas import tpu as pltpu
from jax.experimental.pallas import tpu_sc as plsc

_LANES = 16
_G = 32
_IMAX = 2**31 - 1


def _bcast_max(v):
    pre = plsc.cummax(v)
    suf = jnp.flip(plsc.cummax(jnp.flip(v, 0)), 0)
    return jnp.maximum(pre, suf)


def _bcast_min_i32(v):
    return -_bcast_max(-v)


def _fps_body(N, K, NG, NC, pts_hbm, oidx_hbm, ctr_hbm, rad_hbm, first_hbm,
              out_hbm, xv, yv, zv, dv, ov, cxv, cyv, czv, rv, wv, gmv, actv,
              fv, ob):
    c = lax.axis_index("c")
    s = lax.axis_index("s")
    b = s * NC + c

    pltpu.sync_copy(pts_hbm.at[0, b], xv)
    pltpu.sync_copy(pts_hbm.at[1, b], yv)
    pltpu.sync_copy(pts_hbm.at[2, b], zv)
    pltpu.sync_copy(oidx_hbm.at[b], ov)
    pltpu.sync_copy(ctr_hbm.at[0, b], cxv)
    pltpu.sync_copy(ctr_hbm.at[1, b], cyv)
    pltpu.sync_copy(ctr_hbm.at[2, b], czv)
    pltpu.sync_copy(rad_hbm.at[b], rv)
    pltpu.sync_copy(first_hbm, fv)

    lane = lax.broadcasted_iota(jnp.int32, (_LANES,), 0)
    out_mask = lane < 3
    lane0 = lane == 0

    def init_d(i, carry):
        dv[pl.ds(i * _LANES, _LANES)] = jnp.full((_LANES,), 1e10, jnp.float32)
        return carry

    lax.fori_loop(0, N // _LANES, init_d, 0, unroll=8)

    def init_g(i, carry):
        gmv[pl.ds(i * _LANES, _LANES)] = jnp.full((_LANES,), 1e10, jnp.float32)
        wv[pl.ds(i * _LANES, _LANES)] = jnp.full((_LANES,), 3e38, jnp.float32)
        return carry

    lax.fori_loop(0, NG // _LANES, init_g, 0, unroll=4)

    bvec = jnp.full((_LANES,), b, dtype=jnp.int32)
    fvec = plsc.load_gather(fv, [bvec])
    lx = plsc.load_gather(xv, [fvec])
    ly = plsc.load_gather(yv, [fvec])
    lz = plsc.load_gather(zv, [fvec])
    out0 = jnp.where(lane == 0, lx, jnp.where(lane == 1, ly, lz))
    plsc.store_scatter(ob, [lane], out0, mask=out_mask)

    def step(t, carry):
        clx, cly, clz = carry

        def ph1(i, offv):
            gbase = i * _LANES
            cx = cxv[pl.ds(gbase, _LANES)]
            cy = cyv[pl.ds(gbase, _LANES)]
            cz = czv[pl.ds(gbase, _LANES)]
            w = wv[pl.ds(gbase, _LANES)]
            dcx = cx - clx
            dcy = cy - cly
            dcz = cz - clz
            dc = (dcx * dcx + dcz * dcz) + dcy * dcy
            act = dc < w
            pos = offv + plsc.cumsum(jnp.where(act, 1, 0)) - 1
            plsc.store_scatter(actv, [pos], gbase + lane, mask=act)
            return offv + plsc.all_reduce_population_count(act)

        offv = plsc.parallel_loop(
            0, NG // _LANES, 1, unroll=4,
            carry=jnp.zeros((_LANES,), jnp.int32))(ph1)
        na = jnp.max(offv)

        def ph2(j):
            gidv = plsc.load_gather(actv, [jnp.full((_LANES,), j, jnp.int32)])
            basev = gidv * _G + lane
            gm = jnp.full((_LANES,), -1.0, jnp.float32)
            for u in range(_G // _LANES):
                idxv = basev + u * _LANES
                dx = plsc.load_gather(xv, [idxv]) - clx
                dy = plsc.load_gather(yv, [idxv]) - cly
                dz = plsc.load_gather(zv, [idxv]) - clz
                dist = (dx * dx + dz * dz) + dy * dy
                d = jnp.minimum(plsc.load_gather(dv, [idxv]), dist)
                plsc.store_scatter(dv, [idxv], d)
                gm = jnp.maximum(gm, d)
            gmb = _bcast_max(gm)
            yi = jnp.int32(0x5F3759DF) - (plsc.bitcast(gmb, jnp.int32) >> 1)
            y0 = plsc.bitcast(yi, jnp.float32)
            y1 = y0 * (1.5 - 0.5 * gmb * y0 * y0)
            y2 = y1 * (1.5 - 0.5 * gmb * y1 * y1)
            sq = jnp.maximum(gmb * y2 * 1.0001, 1.2e-19)
            rg = plsc.load_gather(rv, [gidv])
            w = (rg + sq) * (rg + sq) * 1.0001
            plsc.store_scatter(gmv, [gidv], gmb, mask=lane0)
            plsc.store_scatter(wv, [gidv], w, mask=lane0)

        plsc.parallel_loop(0, na, 1, unroll=2)(ph2)

        def ph3(i, rm):
            return jnp.maximum(rm, gmv[pl.ds(i * _LANES, _LANES)])

        rm = lax.fori_loop(0, NG // _LANES, ph3,
                           jnp.full((_LANES,), -1.0, jnp.float32), unroll=4)
        mv = _bcast_max(rm)

        def ph3b(i, offv):
            gbase = i * _LANES
            cand = gmv[pl.ds(gbase, _LANES)] == mv
            pos = offv + plsc.cumsum(jnp.where(cand, 1, 0)) - 1
            plsc.store_scatter(actv, [pos], gbase + lane, mask=cand)
            return offv + plsc.all_reduce_population_count(cand)

        offv = lax.fori_loop(0, NG // _LANES, ph3b,
                             jnp.zeros((_LANES,), jnp.int32))
        nc = jnp.max(offv)

        def ph3c(j, best):
            bo, bp = best
            gidv = plsc.load_gather(actv, [jnp.full((_LANES,), j, jnp.int32)])
            basev = gidv * _G + lane
            o_lane = jnp.full((_LANES,), _IMAX, jnp.int32)
            p_lane = jnp.full((_LANES,), _IMAX, jnp.int32)
            for u in range(_G // _LANES):
                idxv = basev + u * _LANES
                hit = plsc.load_gather(dv, [idxv]) == mv
                oo = jnp.where(hit, plsc.load_gather(ov, [idxv]), _IMAX)
                take = oo < o_lane
                o_lane = jnp.where(take, oo, o_lane)
                p_lane = jnp.where(take, idxv, p_lane)
            obst = _bcast_min_i32(o_lane)
            pbst = _bcast_min_i32(jnp.where(o_lane == obst, p_lane, _IMAX))
            take = obst < bo
            return (jnp.where(take, obst, bo), jnp.where(take, pbst, bp))

        _, bestp = lax.fori_loop(
            0, nc, ph3c,
            (jnp.full((_LANES,), _IMAX, jnp.int32), jnp.full((_LANES,), _IMAX, jnp.int32)))

        nlx = plsc.load_gather(xv, [bestp])
        nly = plsc.load_gather(yv, [bestp])
        nlz = plsc.load_gather(zv, [bestp])
        outv = jnp.where(lane == 0, nlx, jnp.where(lane == 1, nly, nlz))
        plsc.store_scatter(ob, [3 * (t + 1) + lane], outv, mask=out_mask)
        return nlx, nly, nlz

    lax.fori_loop(0, K - 1, step, (lx, ly, lz))
    pltpu.sync_copy(ob, out_hbm.at[b])


def _morton(q):
    def spread(x):
        x = (x | (x << 16)) & 0x030000FF
        x = (x | (x << 8)) & 0x0300F00F
        x = (x | (x << 4)) & 0x030C30C3
        x = (x | (x << 2)) & 0x09249249
        return x

    return (spread(q[..., 0]) << 2) | (spread(q[..., 1]) << 1) | spread(q[..., 2])


def kernel(points):
    B, N, D = points.shape
    K = min(1024, N)
    NG = N // _G
    assert D == 3, "kernel specialized to 3-D points"

    info = plsc.get_sparse_core_info()
    NC, NS = info.num_cores, info.num_subcores
    assert NC * NS == B, (NC, NS, B)

    lo = jnp.min(points, axis=1, keepdims=True)
    hi = jnp.max(points, axis=1, keepdims=True)
    q = jnp.clip(((points - lo) / (hi - lo + 1e-30) * 1024).astype(jnp.int32),
                 0, 1023)
    perm = jnp.argsort(_morton(q), axis=-1).astype(jnp.int32)
    pts_t = jnp.take_along_axis(jnp.transpose(points, (2, 0, 1)),
                                perm[None, :, :], axis=2)

    grp_t = pts_t.reshape(3, B, NG, _G)
    ctr = grp_t.mean(axis=3)
    diff = grp_t - ctr[..., None]
    rad = jnp.sqrt((diff * diff).sum(axis=0).max(axis=-1))
    rad = rad * 1.001 + 1e-30

    first = jax.random.randint(jax.random.key(1), (B,), 0, N).astype(jnp.int32)
    first_p = jnp.argmax(perm == first[:, None], axis=-1).astype(jnp.int32)

    mesh = plsc.VectorSubcoreMesh(core_axis_name="c", subcore_axis_name="s")
    fps = pl.kernel(
        functools.partial(_fps_body, N, K, NG, NC),
        mesh=mesh,
        compiler_params=pltpu.CompilerParams(needs_layout_passes=False),
        out_type=jax.ShapeDtypeStruct((B, K * 3), jnp.float32),
        scratch_types=[
            pltpu.VMEM((N,), jnp.float32),
            pltpu.VMEM((N,), jnp.float32),
            pltpu.VMEM((N,), jnp.float32),
            pltpu.VMEM((N,), jnp.float32),
            pltpu.VMEM((N,), jnp.int32),
            pltpu.VMEM((NG,), jnp.float32),
            pltpu.VMEM((NG,), jnp.float32),
            pltpu.VMEM((NG,), jnp.float32),
            pltpu.VMEM((NG,), jnp.float32),
            pltpu.VMEM((NG,), jnp.float32),
            pltpu.VMEM((NG,), jnp.float32),
            pltpu.VMEM((NG + _LANES,), jnp.int32),
            pltpu.VMEM((B,), jnp.int32),
            pltpu.VMEM((K * 3,), jnp.float32),
        ],
    )
    out = fps(pts_t, perm, ctr, rad, first_p)
    return out.reshape(B, K, 3)


# speedup vs baseline: 38.0538x; 1.0006x over previous
import functools

import jax
import jax.numpy as jnp
from jax import lax
from jax.experimental import pallas as pl
from jax.experimental.pallas import tpu as pltpu
from jax.experimental.pallas import tpu_sc as plsc

_LANES = 16
_G = 32
_IMAX = 2**31 - 1


def _bcast_max(v):
    pre = plsc.cummax(v)
    suf = jnp.flip(plsc.cummax(jnp.flip(v, 0)), 0)
    return jnp.maximum(pre, suf)


def _bcast_min_i32(v):
    return -_bcast_max(-v)


def _fps_body(N, K, NG, NC, pts_hbm, oidx_hbm, ctr_hbm, rad_hbm, first_hbm,
              out_hbm, xv, yv, zv, dv, ov, cxv, cyv, czv, rv, wv, gmv, glm,
              actv, fv, ob):
    c = lax.axis_index("c")
    s = lax.axis_index("s")
    b = s * NC + c

    pltpu.sync_copy(pts_hbm.at[0, b], xv)
    pltpu.sync_copy(pts_hbm.at[1, b], yv)
    pltpu.sync_copy(pts_hbm.at[2, b], zv)
    pltpu.sync_copy(oidx_hbm.at[b], ov)
    pltpu.sync_copy(ctr_hbm.at[0, b], cxv)
    pltpu.sync_copy(ctr_hbm.at[1, b], cyv)
    pltpu.sync_copy(ctr_hbm.at[2, b], czv)
    pltpu.sync_copy(rad_hbm.at[b], rv)
    pltpu.sync_copy(first_hbm, fv)

    lane = lax.broadcasted_iota(jnp.int32, (_LANES,), 0)
    out_mask = lane < 3
    lane0 = lane == 0

    def init_d(i, carry):
        dv[pl.ds(i * _LANES, _LANES)] = jnp.full((_LANES,), 1e10, jnp.float32)
        return carry

    lax.fori_loop(0, N // _LANES, init_d, 0, unroll=8)

    def init_g(i, carry):
        gmv[pl.ds(i * _LANES, _LANES)] = jnp.full((_LANES,), 1e10, jnp.float32)
        wv[pl.ds(i * _LANES, _LANES)] = jnp.full((_LANES,), 3e38, jnp.float32)
        actv[pl.ds(i * _LANES, _LANES)] = jnp.zeros((_LANES,), jnp.int32)
        return carry

    lax.fori_loop(0, NG // _LANES, init_g, 0, unroll=4)
    actv[pl.ds(NG, _LANES)] = jnp.zeros((_LANES,), jnp.int32)

    def init_glm(i, carry):
        glm[pl.ds(i * _LANES, _LANES)] = jnp.full((_LANES,), 1e10, jnp.float32)
        return carry

    lax.fori_loop(0, NG, init_glm, 0, unroll=8)

    bvec = jnp.full((_LANES,), b, dtype=jnp.int32)
    fvec = plsc.load_gather(fv, [bvec])
    lx = plsc.load_gather(xv, [fvec])
    ly = plsc.load_gather(yv, [fvec])
    lz = plsc.load_gather(zv, [fvec])
    out0 = jnp.where(lane == 0, lx, jnp.where(lane == 1, ly, lz))
    plsc.store_scatter(ob, [lane], out0, mask=out_mask)

    def step(t, carry):
        clx, cly, clz = carry

        def ph1(i, offv):
            gbase = i * _LANES
            cx = cxv[pl.ds(gbase, _LANES)]
            cy = cyv[pl.ds(gbase, _LANES)]
            cz = czv[pl.ds(gbase, _LANES)]
            w = wv[pl.ds(gbase, _LANES)]
            dcx = cx - clx
            dcy = cy - cly
            dcz = cz - clz
            dc = (dcx * dcx + dcz * dcz) + dcy * dcy
            act = dc < w
            pos = offv + plsc.cumsum(jnp.where(act, 1, 0)) - 1
            plsc.store_scatter(actv, [pos], gbase + lane, mask=act)
            return offv + plsc.all_reduce_population_count(act)

        offv = plsc.parallel_loop(
            0, NG // _LANES, 1, unroll=4,
            carry=jnp.zeros((_LANES,), jnp.int32))(ph1)
        na = jnp.max(offv)

        def ph2a(j):
            gidv = plsc.load_gather(actv, [jnp.full((_LANES,), j, jnp.int32)])
            basev = gidv * _G + lane
            gm = jnp.full((_LANES,), -1.0, jnp.float32)
            for u in range(_G // _LANES):
                idxv = basev + u * _LANES
                dx = plsc.load_gather(xv, [idxv]) - clx
                dy = plsc.load_gather(yv, [idxv]) - cly
                dz = plsc.load_gather(zv, [idxv]) - clz
                dist = (dx * dx + dz * dz) + dy * dy
                d = jnp.minimum(plsc.load_gather(dv, [idxv]), dist)
                plsc.store_scatter(dv, [idxv], d)
                gm = jnp.maximum(gm, d)
            plsc.store_scatter(glm, [gidv * _LANES + lane], gm)

        plsc.parallel_loop(0, na, 1, unroll=2)(ph2a)

        def ph2b(j):
            gidv = plsc.load_gather(actv, [j * _LANES + lane])
            gb = gidv * _LANES
            gm = plsc.load_gather(glm, [gb])
            for l in range(1, _LANES):
                gm = jnp.maximum(gm, plsc.load_gather(glm, [gb + l]))
            yi = jnp.int32(0x5F3759DF) - (plsc.bitcast(gm, jnp.int32) >> 1)
            y0 = plsc.bitcast(yi, jnp.float32)
            y1 = y0 * (1.5 - 0.5 * gm * y0 * y0)
            y2 = y1 * (1.5 - 0.5 * gm * y1 * y1)
            sq = jnp.maximum(gm * y2 * 1.0001, 1.2e-19)
            rg = plsc.load_gather(rv, [gidv])
            w = (rg + sq) * (rg + sq) * 1.0001
            plsc.store_scatter(gmv, [gidv], gm)
            plsc.store_scatter(wv, [gidv], w)

        plsc.parallel_loop(0, (na + _LANES - 1) // _LANES, 1)(ph2b)

        def ph3(i, rm):
            return jnp.maximum(rm, gmv[pl.ds(i * _LANES, _LANES)])

        rm = lax.fori_loop(0, NG // _LANES, ph3,
                           jnp.full((_LANES,), -1.0, jnp.float32), unroll=4)
        mv = _bcast_max(rm)

        def ph3b(i, offv):
            gbase = i * _LANES
            cand = gmv[pl.ds(gbase, _LANES)] == mv
            pos = offv + plsc.cumsum(jnp.where(cand, 1, 0)) - 1
            plsc.store_scatter(actv, [pos], gbase + lane, mask=cand)
            return offv + plsc.all_reduce_population_count(cand)

        offv = lax.fori_loop(0, NG // _LANES, ph3b,
                             jnp.zeros((_LANES,), jnp.int32))
        nc = jnp.max(offv)

        def ph3c(j, best):
            bo, bp = best
            gidv = plsc.load_gather(actv, [jnp.full((_LANES,), j, jnp.int32)])
            basev = gidv * _G + lane
            o_lane = jnp.full((_LANES,), _IMAX, jnp.int32)
            p_lane = jnp.full((_LANES,), _IMAX, jnp.int32)
            for u in range(_G // _LANES):
                idxv = basev + u * _LANES
                hit = plsc.load_gather(dv, [idxv]) == mv
                oo = jnp.where(hit, plsc.load_gather(ov, [idxv]), _IMAX)
                take = oo < o_lane
                o_lane = jnp.where(take, oo, o_lane)
                p_lane = jnp.where(take, idxv, p_lane)
            obst = _bcast_min_i32(o_lane)
            pbst = _bcast_min_i32(jnp.where(o_lane == obst, p_lane, _IMAX))
            take = obst < bo
            return (jnp.where(take, obst, bo), jnp.where(take, pbst, bp))

        _, bestp = lax.fori_loop(
            0, nc, ph3c,
            (jnp.full((_LANES,), _IMAX, jnp.int32), jnp.full((_LANES,), _IMAX, jnp.int32)))

        nlx = plsc.load_gather(xv, [bestp])
        nly = plsc.load_gather(yv, [bestp])
        nlz = plsc.load_gather(zv, [bestp])
        outv = jnp.where(lane == 0, nlx, jnp.where(lane == 1, nly, nlz))
        plsc.store_scatter(ob, [3 * (t + 1) + lane], outv, mask=out_mask)
        return nlx, nly, nlz

    lax.fori_loop(0, K - 1, step, (lx, ly, lz))
    pltpu.sync_copy(ob, out_hbm.at[b])


def _morton(q):
    def spread(x):
        x = (x | (x << 16)) & 0x030000FF
        x = (x | (x << 8)) & 0x0300F00F
        x = (x | (x << 4)) & 0x030C30C3
        x = (x | (x << 2)) & 0x09249249
        return x

    return (spread(q[..., 0]) << 2) | (spread(q[..., 1]) << 1) | spread(q[..., 2])


def kernel(points):
    B, N, D = points.shape
    K = min(1024, N)
    NG = N // _G
    assert D == 3, "kernel specialized to 3-D points"

    info = plsc.get_sparse_core_info()
    NC, NS = info.num_cores, info.num_subcores
    assert NC * NS == B, (NC, NS, B)

    lo = jnp.min(points, axis=1, keepdims=True)
    hi = jnp.max(points, axis=1, keepdims=True)
    q = jnp.clip(((points - lo) / (hi - lo + 1e-30) * 1024).astype(jnp.int32),
                 0, 1023)
    perm = jnp.argsort(_morton(q), axis=-1).astype(jnp.int32)
    pts_t = jnp.take_along_axis(jnp.transpose(points, (2, 0, 1)),
                                perm[None, :, :], axis=2)

    grp_t = pts_t.reshape(3, B, NG, _G)
    ctr = grp_t.mean(axis=3)
    diff = grp_t - ctr[..., None]
    rad = jnp.sqrt((diff * diff).sum(axis=0).max(axis=-1))
    rad = rad * 1.001 + 1e-30

    first = jax.random.randint(jax.random.key(1), (B,), 0, N).astype(jnp.int32)
    first_p = jnp.argmax(perm == first[:, None], axis=-1).astype(jnp.int32)

    mesh = plsc.VectorSubcoreMesh(core_axis_name="c", subcore_axis_name="s")
    fps = pl.kernel(
        functools.partial(_fps_body, N, K, NG, NC),
        mesh=mesh,
        compiler_params=pltpu.CompilerParams(needs_layout_passes=False),
        out_type=jax.ShapeDtypeStruct((B, K * 3), jnp.float32),
        scratch_types=[
            pltpu.VMEM((N,), jnp.float32),
            pltpu.VMEM((N,), jnp.float32),
            pltpu.VMEM((N,), jnp.float32),
            pltpu.VMEM((N,), jnp.float32),
            pltpu.VMEM((N,), jnp.int32),
            pltpu.VMEM((NG,), jnp.float32),
            pltpu.VMEM((NG,), jnp.float32),
            pltpu.VMEM((NG,), jnp.float32),
            pltpu.VMEM((NG,), jnp.float32),
            pltpu.VMEM((NG,), jnp.float32),
            pltpu.VMEM((NG,), jnp.float32),
            pltpu.VMEM((NG * _LANES,), jnp.float32),
            pltpu.VMEM((NG + _LANES,), jnp.int32),
            pltpu.VMEM((B,), jnp.int32),
            pltpu.VMEM((K * 3,), jnp.float32),
        ],
    )
    out = fps(pts_t, perm, ctr, rad, first_p)
    return out.reshape(B, K, 3)
